# Initial kernel scaffold; baseline (speedup 1.0000x reference)
import functools

import jax
import jax.numpy as jnp
from jax import lax
from jax.experimental import pallas as pl
from jax.experimental.pallas import tpu as pltpu
from jax.experimental.pallas import tpu_sc as plsc

H = 150
W = 150
NQ = H * W
EMBED = 256
HEADS = 8
POINTS = 4
QUEUE = 2
HD = EMBED // HEADS
B2H = QUEUE * HEADS
R = POINTS * 4

QB = 512
NQP = 44 * QB
NBLK = NQP // QB

NC = 2
NS = 16
LANES = 16
NW = NC * NS

CQ = 88
CHUNKS_PER_B2H = NQP // CQ
TOTAL_CHUNKS = B2H * CHUNKS_PER_B2H
CHUNKS_PER_W = TOTAL_CHUNKS // NW
GB = 128
NGB = (CQ * R) // GB


def _a1_body(v_ref, wv_ref, bv_ref, tab_ref):
    y = jnp.dot(v_ref[...], wv_ref[...], preferred_element_type=jnp.float32)
    y = y + bv_ref[...]
    for h in range(HEADS):
        tab_ref[h] = y[:, h * HD:(h + 1) * HD]


def _a2_body(v0_ref, q_ref, ref_ref, wsx_ref, wsy_ref, bsx_ref, bsy_ref,
             wa_ref, ba_ref, idx_ref, w_ref):
    qe = jnp.concatenate([v0_ref[...], q_ref[...]], axis=1)
    sox = jnp.dot(qe, wsx_ref[...], preferred_element_type=jnp.float32) + bsx_ref[...]
    soy = jnp.dot(qe, wsy_ref[...], preferred_element_type=jnp.float32) + bsy_ref[...]
    a = jnp.dot(qe, wa_ref[...], preferred_element_type=jnp.float32) + ba_ref[...]
    s = jnp.exp(a)
    gi = lax.broadcasted_iota(jnp.int32, (64, 64), 0) // POINTS
    gj = lax.broadcasted_iota(jnp.int32, (64, 64), 1) // POINTS
    gmat = (gi == gj).astype(jnp.float32)
    denom = jnp.dot(s, gmat, preferred_element_type=jnp.float32)
    attn = s / denom

    refx = ref_ref[:, 0:1]
    refy = ref_ref[:, 1:2]
    x = refx * W + sox - 0.5
    y = refy * H + soy - 0.5
    x0 = jnp.floor(x)
    y0 = jnp.floor(y)
    fx = x - x0
    fy = y - y0
    x0i = x0.astype(jnp.int32)
    y0i = y0.astype(jnp.int32)

    col = lax.broadcasted_iota(jnp.int32, (QB, 64), 1)
    h_ = col // 8
    qq_ = (col // 4) % 2
    b2h_off = (qq_ * HEADS + h_) * NQP

    idx_parts = []
    w_parts = []
    for dx, dy, wc in ((0, 0, (1.0 - fx) * (1.0 - fy)),
                       (1, 0, fx * (1.0 - fy)),
                       (0, 1, (1.0 - fx) * fy),
                       (1, 1, fx * fy)):
        xi = x0i + dx
        yi = y0i + dy
        valid = (xi >= 0) & (xi <= W - 1) & (yi >= 0) & (yi <= H - 1)
        xc = jnp.clip(xi, 0, W - 1)
        yc = jnp.clip(yi, 0, H - 1)
        idx_parts.append(b2h_off + yc * W + xc)
        w_parts.append(wc * attn * valid.astype(jnp.float32))
    idx_ref[...] = jnp.concatenate(idx_parts, axis=1)
    w_ref[...] = jnp.concatenate(w_parts, axis=1)


def _b_body(s0_ref, s1_ref, wo_ref, bo_ref, q_ref, out_ref):
    a = jnp.concatenate([s0_ref[h] for h in range(HEADS)], axis=1)
    b = jnp.concatenate([s1_ref[h] for h in range(HEADS)], axis=1)
    m = (a + b) * 0.5
    out_ref[...] = (jnp.dot(m, wo_ref[...], preferred_element_type=jnp.float32)
                    + bo_ref[...] + q_ref[...])


def _tc_prepare(query2d, value2d, refs2d, W_samp, b_samp, W_attn, b_attn,
                W_val, b_val, *, interpret=False):
    padq = NQP - NQ
    q_pad = jnp.pad(query2d, ((0, padq), (0, 0)))
    v3 = value2d.reshape(QUEUE, NQ, EMBED)
    v_pad = jnp.pad(v3, ((0, 0), (0, padq), (0, 0))).reshape(QUEUE * NQP, EMBED)
    refs_pad = jnp.pad(refs2d, ((0, padq), (0, 0)))

    wv_t = W_val.T
    bv = b_val.reshape(1, EMBED)
    table = pl.pallas_call(
        _a1_body,
        grid=(QUEUE, NBLK),
        in_specs=[
            pl.BlockSpec((QB, EMBED), lambda qq, i: (qq * NBLK + i, 0)),
            pl.BlockSpec((EMBED, EMBED), lambda qq, i: (0, 0)),
            pl.BlockSpec((1, EMBED), lambda qq, i: (0, 0)),
        ],
        out_specs=pl.BlockSpec((HEADS, QB, HD), lambda qq, i: (qq, i, 0)),
        out_shape=jax.ShapeDtypeStruct((B2H, NQP, HD), jnp.float32),
        interpret=interpret,
    )(v_pad, wv_t, bv)

    wsx = W_samp[0::2, :].T
    wsy = W_samp[1::2, :].T
    bsx = b_samp[0::2].reshape(1, 64)
    bsy = b_samp[1::2].reshape(1, 64)
    wa = W_attn.T
    ba = b_attn.reshape(1, 64)

    idx_nat, w_nat = pl.pallas_call(
        _a2_body,
        grid=(NBLK,),
        in_specs=[
            pl.BlockSpec((QB, EMBED), lambda i: (i, 0)),
            pl.BlockSpec((QB, EMBED), lambda i: (i, 0)),
            pl.BlockSpec((QB, 2), lambda i: (i, 0)),
            pl.BlockSpec((2 * EMBED, 64), lambda i: (0, 0)),
            pl.BlockSpec((2 * EMBED, 64), lambda i: (0, 0)),
            pl.BlockSpec((1, 64), lambda i: (0, 0)),
            pl.BlockSpec((1, 64), lambda i: (0, 0)),
            pl.BlockSpec((2 * EMBED, 64), lambda i: (0, 0)),
            pl.BlockSpec((1, 64), lambda i: (0, 0)),
        ],
        out_specs=[
            pl.BlockSpec((QB, 4 * 64), lambda i: (i, 0)),
            pl.BlockSpec((QB, 4 * 64), lambda i: (i, 0)),
        ],
        out_shape=[
            jax.ShapeDtypeStruct((NQP, 4 * 64), jnp.int32),
            jax.ShapeDtypeStruct((NQP, 4 * 64), jnp.float32),
        ],
        interpret=interpret,
    )(v_pad[:NQP], q_pad, refs_pad, wsx, wsy, bsx, bsy, wa, ba)

    idx_all = idx_nat.reshape(NQP, 4, HEADS, QUEUE, POINTS)
    idx_all = idx_all.transpose(3, 2, 0, 1, 4).reshape(B2H * NQP * R)
    w_all = w_nat.reshape(NQP, 4, HEADS, QUEUE, POINTS)
    w_all = w_all.transpose(3, 2, 0, 1, 4).reshape(B2H * NQP * R)
    return table, idx_all, w_all, q_pad


_sc_mesh = plsc.VectorSubcoreMesh(core_axis_name="c", subcore_axis_name="s")


@functools.partial(
    pl.kernel,
    out_type=jax.ShapeDtypeStruct((B2H * NQP * HD,), jnp.float32),
    mesh=_sc_mesh,
    scratch_types=[
        pltpu.VMEM((NGB, GB), jnp.int32),
        pltpu.VMEM((CQ * R,), jnp.float32),
        pltpu.VMEM((CQ * R, HD), jnp.float32),
        pltpu.VMEM((CQ * HD,), jnp.float32),
        pltpu.SemaphoreType.DMA,
    ],
)
def _sc_sample(table_hbm, idx_hbm, w_hbm, out_hbm, idx_v, w_v, rows_v, out_v, sem):
    cid = lax.axis_index("c")
    sid = lax.axis_index("s")
    wid = sid * NC + cid

    def chunk_body(i, carry):
        t = wid * CHUNKS_PER_W + i
        pltpu.sync_copy(idx_hbm.at[pl.ds(t * NGB, NGB)], idx_v)
        pltpu.sync_copy(w_hbm.at[pl.ds(t * (CQ * R), CQ * R)], w_v)
        copies = [
            pltpu.async_copy(table_hbm.at[idx_v.at[b]],
                             rows_v.at[pl.ds(b * GB, GB)], sem)
            for b in range(NGB)
        ]
        for c in copies:
            c.wait()

        def q_body(j, carry2):
            wv = w_v[pl.ds(j * R, LANES)]
            acc0 = jnp.zeros((LANES,), jnp.float32)
            acc1 = jnp.zeros((LANES,), jnp.float32)
            for r in range(R):
                wb = jnp.take(wv, jnp.full((LANES,), r, jnp.int32),
                              mode="promise_in_bounds")
                jr = j * R + r
                acc0 = acc0 + rows_v[jr, pl.ds(0, LANES)] * wb
                acc1 = acc1 + rows_v[jr, pl.ds(LANES, LANES)] * wb
            out_v[pl.ds(j * HD, LANES)] = acc0
            out_v[pl.ds(j * HD + LANES, LANES)] = acc1
            return carry2

        lax.fori_loop(0, CQ, q_body, 0)
        pltpu.sync_copy(out_v, out_hbm.at[pl.ds(t * (CQ * HD), CQ * HD)])
        return carry

    lax.fori_loop(0, CHUNKS_PER_W, chunk_body, 0)


def _tc_finish(out_sc, q_pad, W_out, b_out, *, interpret=False):
    wo_t = W_out.T
    bo = b_out.reshape(1, EMBED)
    out = pl.pallas_call(
        _b_body,
        grid=(NBLK,),
        in_specs=[
            pl.BlockSpec((HEADS, QB, HD), lambda i: (0, i, 0)),
            pl.BlockSpec((HEADS, QB, HD), lambda i: (1, i, 0)),
            pl.BlockSpec((EMBED, EMBED), lambda i: (0, 0)),
            pl.BlockSpec((1, EMBED), lambda i: (0, 0)),
            pl.BlockSpec((QB, EMBED), lambda i: (i, 0)),
        ],
        out_specs=pl.BlockSpec((QB, EMBED), lambda i: (i, 0)),
        out_shape=jax.ShapeDtypeStruct((NQP, EMBED), jnp.float32),
        interpret=interpret,
    )(out_sc, out_sc, wo_t, bo, q_pad)
    return out


def kernel(query, value, reference_points, spatial_shapes, level_start_index,
           W_samp, b_samp, W_attn, b_attn, W_val, b_val, W_out, b_out):
    query2d = query[0]
    value2d = value[0]
    refs2d = reference_points[0, :, 0, :]

    table, idx_all, w_all, q_pad = _tc_prepare(
        query2d, value2d, refs2d, W_samp, b_samp, W_attn, b_attn, W_val, b_val)

    table2d = table.reshape(B2H * NQP, HD)
    idx2d = idx_all.reshape(B2H * NQP * R // GB, GB)
    out_sc = _sc_sample(table2d, idx2d, w_all)

    out_sc = out_sc.reshape(B2H, NQP, HD)
    out = _tc_finish(out_sc, q_pad, W_out, b_out)
    return out[:NQ].reshape(1, NQ, EMBED)


# speedup vs baseline: 692.8386x; 692.8386x over previous
import functools

import jax
import jax.numpy as jnp
from jax import lax
from jax.experimental import pallas as pl
from jax.experimental.pallas import tpu as pltpu
from jax.experimental.pallas import tpu_sc as plsc

H = 150
W = 150
NQ = H * W
EMBED = 256
HEADS = 8
POINTS = 4
QUEUE = 2
HD = EMBED // HEADS
B2H = QUEUE * HEADS
R = POINTS * 4

QB = 512
NQP = 44 * QB
NBLK = NQP // QB

NC = 2
NS = 16
LANES = 16
NW = NC * NS

CQ = 88
CHUNKS_PER_B2H = NQP // CQ
TOTAL_CHUNKS = B2H * CHUNKS_PER_B2H
CHUNKS_PER_W = TOTAL_CHUNKS // NW
GB = 88
NGB = (CQ * R) // GB


def _a1_body(v_ref, wv_ref, bv_ref, tab_ref):
    y = jnp.dot(v_ref[...], wv_ref[...], preferred_element_type=jnp.float32)
    y = y + bv_ref[...]
    for h in range(HEADS):
        tab_ref[h] = y[:, h * HD:(h + 1) * HD]


def _a2_body(v0_ref, q_ref, ref_ref, wsx_ref, wsy_ref, bsx_ref, bsy_ref,
             wa_ref, ba_ref, idx_ref, w_ref):
    qe = jnp.concatenate([v0_ref[...], q_ref[...]], axis=1)
    sox = jnp.dot(qe, wsx_ref[...], preferred_element_type=jnp.float32) + bsx_ref[...]
    soy = jnp.dot(qe, wsy_ref[...], preferred_element_type=jnp.float32) + bsy_ref[...]
    a = jnp.dot(qe, wa_ref[...], preferred_element_type=jnp.float32) + ba_ref[...]
    s = jnp.exp(a)
    gi = lax.broadcasted_iota(jnp.int32, (64, 64), 0) // POINTS
    gj = lax.broadcasted_iota(jnp.int32, (64, 64), 1) // POINTS
    gmat = (gi == gj).astype(jnp.float32)
    denom = jnp.dot(s, gmat, preferred_element_type=jnp.float32)
    attn = s / denom

    refx = ref_ref[:, 0:1]
    refy = ref_ref[:, 1:2]
    x = refx * W + sox - 0.5
    y = refy * H + soy - 0.5
    x0 = jnp.floor(x)
    y0 = jnp.floor(y)
    fx = x - x0
    fy = y - y0
    x0i = x0.astype(jnp.int32)
    y0i = y0.astype(jnp.int32)

    col = lax.broadcasted_iota(jnp.int32, (QB, 64), 1)
    h_ = col // 8
    qq_ = (col // 4) % 2
    b2h_off = (qq_ * HEADS + h_) * NQP

    idx_parts = []
    w_parts = []
    for dx, dy, wc in ((0, 0, (1.0 - fx) * (1.0 - fy)),
                       (1, 0, fx * (1.0 - fy)),
                       (0, 1, (1.0 - fx) * fy),
                       (1, 1, fx * fy)):
        xi = x0i + dx
        yi = y0i + dy
        valid = (xi >= 0) & (xi <= W - 1) & (yi >= 0) & (yi <= H - 1)
        xc = jnp.clip(xi, 0, W - 1)
        yc = jnp.clip(yi, 0, H - 1)
        idx_parts.append(b2h_off + yc * W + xc)
        w_parts.append(wc * attn * valid.astype(jnp.float32))
    idx_ref[...] = jnp.concatenate(idx_parts, axis=1)
    w_ref[...] = jnp.concatenate(w_parts, axis=1)


def _b_body(s0_ref, s1_ref, wo_ref, bo_ref, q_ref, out_ref):
    a = jnp.concatenate([s0_ref[h] for h in range(HEADS)], axis=1)
    b = jnp.concatenate([s1_ref[h] for h in range(HEADS)], axis=1)
    m = (a + b) * 0.5
    out_ref[...] = (jnp.dot(m, wo_ref[...], preferred_element_type=jnp.float32)
                    + bo_ref[...] + q_ref[...])


def _tc_prepare(query2d, value2d, refs2d, W_samp, b_samp, W_attn, b_attn,
                W_val, b_val, *, interpret=False):
    padq = NQP - NQ
    q_pad = jnp.pad(query2d, ((0, padq), (0, 0)))
    v3 = value2d.reshape(QUEUE, NQ, EMBED)
    v_pad = jnp.pad(v3, ((0, 0), (0, padq), (0, 0))).reshape(QUEUE * NQP, EMBED)
    refs_pad = jnp.pad(refs2d, ((0, padq), (0, 0)))

    wv_t = W_val.T
    bv = b_val.reshape(1, EMBED)
    table = pl.pallas_call(
        _a1_body,
        grid=(QUEUE, NBLK),
        in_specs=[
            pl.BlockSpec((QB, EMBED), lambda qq, i: (qq * NBLK + i, 0)),
            pl.BlockSpec((EMBED, EMBED), lambda qq, i: (0, 0)),
            pl.BlockSpec((1, EMBED), lambda qq, i: (0, 0)),
        ],
        out_specs=pl.BlockSpec((HEADS, QB, HD), lambda qq, i: (qq, i, 0)),
        out_shape=jax.ShapeDtypeStruct((B2H, NQP, HD), jnp.float32),
        interpret=interpret,
    )(v_pad, wv_t, bv)

    wsx = W_samp[0::2, :].T
    wsy = W_samp[1::2, :].T
    bsx = b_samp[0::2].reshape(1, 64)
    bsy = b_samp[1::2].reshape(1, 64)
    wa = W_attn.T
    ba = b_attn.reshape(1, 64)

    idx_nat, w_nat = pl.pallas_call(
        _a2_body,
        grid=(NBLK,),
        in_specs=[
            pl.BlockSpec((QB, EMBED), lambda i: (i, 0)),
            pl.BlockSpec((QB, EMBED), lambda i: (i, 0)),
            pl.BlockSpec((QB, 2), lambda i: (i, 0)),
            pl.BlockSpec((2 * EMBED, 64), lambda i: (0, 0)),
            pl.BlockSpec((2 * EMBED, 64), lambda i: (0, 0)),
            pl.BlockSpec((1, 64), lambda i: (0, 0)),
            pl.BlockSpec((1, 64), lambda i: (0, 0)),
            pl.BlockSpec((2 * EMBED, 64), lambda i: (0, 0)),
            pl.BlockSpec((1, 64), lambda i: (0, 0)),
        ],
        out_specs=[
            pl.BlockSpec((QB, 4 * 64), lambda i: (i, 0)),
            pl.BlockSpec((QB, 4 * 64), lambda i: (i, 0)),
        ],
        out_shape=[
            jax.ShapeDtypeStruct((NQP, 4 * 64), jnp.int32),
            jax.ShapeDtypeStruct((NQP, 4 * 64), jnp.float32),
        ],
        interpret=interpret,
    )(v_pad[:NQP], q_pad, refs_pad, wsx, wsy, bsx, bsy, wa, ba)

    idx_all = idx_nat.reshape(NQP, 4, HEADS, QUEUE, POINTS)
    idx_all = idx_all.transpose(3, 2, 0, 1, 4).reshape(B2H * NQP * R)
    w_all = w_nat.reshape(NQP, 4, HEADS, QUEUE, POINTS)
    w_all = w_all.transpose(3, 2, 0, 1, 4).reshape(B2H * NQP * R)
    return table, idx_all, w_all, q_pad


def _bcast_lane(vec, r):
    idx = jnp.full((LANES, 1), r, jnp.int32)
    dn = lax.GatherDimensionNumbers(offset_dims=(), collapsed_slice_dims=(0,),
                                    start_index_map=(0,))
    return lax.gather(vec, idx, dn, (1,),
                      mode=lax.GatherScatterMode.PROMISE_IN_BOUNDS)


def _sc_body(table_hbm, idx_hbm, w_hbm, out_hbm, idx_v, w_v, rows_v, out_v, sem):
    cid = lax.axis_index("c")
    sid = lax.axis_index("s")
    wid = sid * NC + cid

    def chunk_body(i, carry):
        t = wid * CHUNKS_PER_W + i
        pltpu.sync_copy(idx_hbm.at[pl.ds(t * NGB, NGB)], idx_v)
        pltpu.sync_copy(w_hbm.at[pl.ds(t * (CQ * R), CQ * R)], w_v)
        copies = [
            pltpu.async_copy(table_hbm.at[idx_v.at[b]],
                             rows_v.at[pl.ds(b * GB, GB)], sem)
            for b in range(NGB)
        ]
        for c in copies:
            c.wait()

        def q_body(j, carry2):
            wv = w_v[pl.ds(j * R, LANES)]
            acc0 = jnp.zeros((LANES,), jnp.float32)
            acc1 = jnp.zeros((LANES,), jnp.float32)
            for r in range(R):
                wb = _bcast_lane(wv, r)
                jr = j * R + r
                acc0 = acc0 + rows_v[jr, pl.ds(0, LANES)] * wb
                acc1 = acc1 + rows_v[jr, pl.ds(LANES, LANES)] * wb
            out_v[pl.ds(j * HD, LANES)] = acc0
            out_v[pl.ds(j * HD + LANES, LANES)] = acc1
            return carry2

        lax.fori_loop(0, CQ, q_body, 0)
        pltpu.sync_copy(out_v, out_hbm.at[pl.ds(t * (CQ * HD), CQ * HD)])
        return carry

    lax.fori_loop(0, CHUNKS_PER_W, chunk_body, 0)


@functools.lru_cache(maxsize=1)
def _sc_sample_fn():
    mesh = plsc.VectorSubcoreMesh(core_axis_name="c", subcore_axis_name="s")
    return pl.kernel(
        _sc_body,
        out_type=jax.ShapeDtypeStruct((B2H * NQP * HD,), jnp.float32),
        mesh=mesh,
        scratch_types=[
            pltpu.VMEM((NGB, GB), jnp.int32),
            pltpu.VMEM((CQ * R,), jnp.float32),
            pltpu.VMEM((CQ * R, HD), jnp.float32),
            pltpu.VMEM((CQ * HD,), jnp.float32),
            pltpu.SemaphoreType.DMA,
        ],
        compiler_params=pltpu.CompilerParams(use_tc_tiling_on_sc=False),
    )


def _tc_finish(out_sc, q_pad, W_out, b_out, *, interpret=False):
    wo_t = W_out.T
    bo = b_out.reshape(1, EMBED)
    out = pl.pallas_call(
        _b_body,
        grid=(NBLK,),
        in_specs=[
            pl.BlockSpec((HEADS, QB, HD), lambda i: (0, i, 0)),
            pl.BlockSpec((HEADS, QB, HD), lambda i: (1, i, 0)),
            pl.BlockSpec((EMBED, EMBED), lambda i: (0, 0)),
            pl.BlockSpec((1, EMBED), lambda i: (0, 0)),
            pl.BlockSpec((QB, EMBED), lambda i: (i, 0)),
        ],
        out_specs=pl.BlockSpec((QB, EMBED), lambda i: (i, 0)),
        out_shape=jax.ShapeDtypeStruct((NQP, EMBED), jnp.float32),
        interpret=interpret,
    )(out_sc, out_sc, wo_t, bo, q_pad)
    return out


def kernel(query, value, reference_points, spatial_shapes, level_start_index,
           W_samp, b_samp, W_attn, b_attn, W_val, b_val, W_out, b_out):
    query2d = query[0]
    value2d = value[0]
    refs2d = reference_points[0, :, 0, :]

    table, idx_all, w_all, q_pad = _tc_prepare(
        query2d, value2d, refs2d, W_samp, b_samp, W_attn, b_attn, W_val, b_val)

    table2d = table.reshape(B2H * NQP, HD)
    idx2d = idx_all.reshape(B2H * NQP * R // GB, GB)
    out_sc = _sc_sample_fn()(table2d, idx2d, w_all)

    out_sc = out_sc.reshape(B2H, NQP, HD)
    out = _tc_finish(out_sc, q_pad, W_out, b_out)
    return out[:NQ].reshape(1, NQ, EMBED)


# speedup vs baseline: 1709.5923x; 2.4675x over previous
import functools

import jax
import jax.numpy as jnp
from jax import lax
from jax.experimental import pallas as pl
from jax.experimental.pallas import tpu as pltpu
from jax.experimental.pallas import tpu_sc as plsc

H = 150
W = 150
NQ = H * W
EMBED = 256
HEADS = 8
POINTS = 4
QUEUE = 2
HD = EMBED // HEADS
B2H = QUEUE * HEADS
R = POINTS * 4

QB = 512
NQP = 44 * QB
NBLK = NQP // QB

NC = 2
NS = 16
LANES = 16
NW = NC * NS

CQ = 88
CHUNKS_PER_B2H = NQP // CQ
TOTAL_CHUNKS = B2H * CHUNKS_PER_B2H
CHUNKS_PER_W = TOTAL_CHUNKS // NW
GB = 88
NGB = (CQ * R) // GB


def _a1_body(v_ref, wv_ref, bv_ref, tab_ref):
    y = jnp.dot(v_ref[...], wv_ref[...], preferred_element_type=jnp.float32)
    y = y + bv_ref[...]
    for h in range(HEADS):
        tab_ref[h] = y[:, h * HD:(h + 1) * HD]


def _a2_body(v0_ref, q_ref, ref_ref, wsx_ref, wsy_ref, bsx_ref, bsy_ref,
             wa_ref, ba_ref, idx_ref, w_ref):
    qe = jnp.concatenate([v0_ref[...], q_ref[...]], axis=1)
    sox = jnp.dot(qe, wsx_ref[...], preferred_element_type=jnp.float32) + bsx_ref[...]
    soy = jnp.dot(qe, wsy_ref[...], preferred_element_type=jnp.float32) + bsy_ref[...]
    a = jnp.dot(qe, wa_ref[...], preferred_element_type=jnp.float32) + ba_ref[...]
    s = jnp.exp(a)
    gi = lax.broadcasted_iota(jnp.int32, (64, 64), 0) // POINTS
    gj = lax.broadcasted_iota(jnp.int32, (64, 64), 1) // POINTS
    gmat = (gi == gj).astype(jnp.float32)
    denom = jnp.dot(s, gmat, preferred_element_type=jnp.float32)
    attn = s / denom

    refx = ref_ref[:, 0:1]
    refy = ref_ref[:, 1:2]
    x = refx * W + sox - 0.5
    y = refy * H + soy - 0.5
    x0 = jnp.floor(x)
    y0 = jnp.floor(y)
    fx = x - x0
    fy = y - y0
    x0i = x0.astype(jnp.int32)
    y0i = y0.astype(jnp.int32)

    col = lax.broadcasted_iota(jnp.int32, (QB, 64), 1)
    b2h_off = (col // POINTS) * NQP

    idx_parts = []
    w_parts = []
    for dx, dy, wc in ((0, 0, (1.0 - fx) * (1.0 - fy)),
                       (1, 0, fx * (1.0 - fy)),
                       (0, 1, (1.0 - fx) * fy),
                       (1, 1, fx * fy)):
        xi = x0i + dx
        yi = y0i + dy
        valid = (xi >= 0) & (xi <= W - 1) & (yi >= 0) & (yi <= H - 1)
        xc = jnp.clip(xi, 0, W - 1)
        yc = jnp.clip(yi, 0, H - 1)
        idx_parts.append(b2h_off + yc * W + xc)
        w_parts.append(wc * attn * valid.astype(jnp.float32))
    for b in range(B2H):
        sl = slice(b * POINTS, (b + 1) * POINTS)
        idx_ref[b] = jnp.concatenate([p_[:, sl] for p_ in idx_parts], axis=1)
        w_ref[b] = jnp.concatenate([p_[:, sl] for p_ in w_parts], axis=1)


def _b_body(s0_ref, s1_ref, wo_ref, bo_ref, q_ref, out_ref):
    a = jnp.concatenate([s0_ref[h] for h in range(HEADS)], axis=1)
    b = jnp.concatenate([s1_ref[h] for h in range(HEADS)], axis=1)
    m = (a + b) * 0.5
    out_ref[...] = (jnp.dot(m, wo_ref[...], preferred_element_type=jnp.float32)
                    + bo_ref[...] + q_ref[...])


def _tc_prepare(query2d, value2d, refs2d, W_samp, b_samp, W_attn, b_attn,
                W_val, b_val, *, interpret=False):
    padq = NQP - NQ
    q_pad = jnp.pad(query2d, ((0, padq), (0, 0)))
    v3 = value2d.reshape(QUEUE, NQ, EMBED)
    v_pad = jnp.pad(v3, ((0, 0), (0, padq), (0, 0))).reshape(QUEUE * NQP, EMBED)
    refs_pad = jnp.pad(refs2d, ((0, padq), (0, 0)))

    wv_t = W_val.T
    bv = b_val.reshape(1, EMBED)
    table = pl.pallas_call(
        _a1_body,
        grid=(QUEUE, NBLK),
        in_specs=[
            pl.BlockSpec((QB, EMBED), lambda qq, i: (qq * NBLK + i, 0)),
            pl.BlockSpec((EMBED, EMBED), lambda qq, i: (0, 0)),
            pl.BlockSpec((1, EMBED), lambda qq, i: (0, 0)),
        ],
        out_specs=pl.BlockSpec((HEADS, QB, HD), lambda qq, i: (qq, i, 0)),
        out_shape=jax.ShapeDtypeStruct((B2H, NQP, HD), jnp.float32),
        interpret=interpret,
    )(v_pad, wv_t, bv)

    cols = jnp.arange(64, dtype=jnp.int32)
    qq_c = cols // 32
    h_c = (cols // POINTS) % HEADS
    p_c = cols % POINTS
    perm = (h_c * QUEUE + qq_c) * POINTS + p_c
    wsx = W_samp[2 * perm, :].T
    wsy = W_samp[2 * perm + 1, :].T
    bsx = b_samp[2 * perm].reshape(1, 64)
    bsy = b_samp[2 * perm + 1].reshape(1, 64)
    wa = W_attn[perm, :].T
    ba = b_attn[perm].reshape(1, 64)

    idx_nat, w_nat = pl.pallas_call(
        _a2_body,
        grid=(NBLK,),
        in_specs=[
            pl.BlockSpec((QB, EMBED), lambda i: (i, 0)),
            pl.BlockSpec((QB, EMBED), lambda i: (i, 0)),
            pl.BlockSpec((QB, 2), lambda i: (i, 0)),
            pl.BlockSpec((2 * EMBED, 64), lambda i: (0, 0)),
            pl.BlockSpec((2 * EMBED, 64), lambda i: (0, 0)),
            pl.BlockSpec((1, 64), lambda i: (0, 0)),
            pl.BlockSpec((1, 64), lambda i: (0, 0)),
            pl.BlockSpec((2 * EMBED, 64), lambda i: (0, 0)),
            pl.BlockSpec((1, 64), lambda i: (0, 0)),
        ],
        out_specs=[
            pl.BlockSpec((B2H, QB, R), lambda i: (0, i, 0)),
            pl.BlockSpec((B2H, QB, R), lambda i: (0, i, 0)),
        ],
        out_shape=[
            jax.ShapeDtypeStruct((B2H, NQP, R), jnp.int32),
            jax.ShapeDtypeStruct((B2H, NQP, R), jnp.float32),
        ],
        interpret=interpret,
    )(v_pad[:NQP], q_pad, refs_pad, wsx, wsy, bsx, bsy, wa, ba)

    idx_all = idx_nat.reshape(B2H * NQP * R)
    w_all = w_nat.reshape(B2H * NQP * R)
    return table, idx_all, w_all, q_pad


def _bcast_lane(vec, r):
    idx = jnp.full((LANES, 1), r, jnp.int32)
    dn = lax.GatherDimensionNumbers(offset_dims=(), collapsed_slice_dims=(0,),
                                    start_index_map=(0,))
    return lax.gather(vec, idx, dn, (1,),
                      mode=lax.GatherScatterMode.PROMISE_IN_BOUNDS)


def _sc_body(table_hbm, idx_hbm, w_hbm, out_hbm, idx_v, w_v, rows_v, out_v, sem):
    cid = lax.axis_index("c")
    sid = lax.axis_index("s")
    wid = sid * NC + cid

    def chunk_body(i, carry):
        t = wid * CHUNKS_PER_W + i
        pltpu.sync_copy(idx_hbm.at[pl.ds(t * NGB, NGB)], idx_v)
        pltpu.sync_copy(w_hbm.at[pl.ds(t * (CQ * R), CQ * R)], w_v)
        copies = [
            pltpu.async_copy(table_hbm.at[idx_v.at[b]],
                             rows_v.at[pl.ds(b * GB, GB)], sem)
            for b in range(NGB)
        ]
        for c in copies:
            c.wait()

        def q_body(j, carry2):
            wv = w_v[pl.ds(j * R, LANES)]
            acc0 = jnp.zeros((LANES,), jnp.float32)
            acc1 = jnp.zeros((LANES,), jnp.float32)
            for r in range(R):
                wb = _bcast_lane(wv, r)
                jr = j * R + r
                acc0 = acc0 + rows_v[jr, pl.ds(0, LANES)] * wb
                acc1 = acc1 + rows_v[jr, pl.ds(LANES, LANES)] * wb
            out_v[pl.ds(j * HD, LANES)] = acc0
            out_v[pl.ds(j * HD + LANES, LANES)] = acc1
            return carry2

        lax.fori_loop(0, CQ, q_body, 0)
        pltpu.sync_copy(out_v, out_hbm.at[pl.ds(t * (CQ * HD), CQ * HD)])
        return carry

    lax.fori_loop(0, CHUNKS_PER_W, chunk_body, 0)


@functools.lru_cache(maxsize=1)
def _sc_sample_fn():
    mesh = plsc.VectorSubcoreMesh(core_axis_name="c", subcore_axis_name="s")
    return pl.kernel(
        _sc_body,
        out_type=jax.ShapeDtypeStruct((B2H * NQP * HD,), jnp.float32),
        mesh=mesh,
        scratch_types=[
            pltpu.VMEM((NGB, GB), jnp.int32),
            pltpu.VMEM((CQ * R,), jnp.float32),
            pltpu.VMEM((CQ * R, HD), jnp.float32),
            pltpu.VMEM((CQ * HD,), jnp.float32),
            pltpu.SemaphoreType.DMA,
        ],
        compiler_params=pltpu.CompilerParams(use_tc_tiling_on_sc=False),
    )


def _tc_finish(out_sc, q_pad, W_out, b_out, *, interpret=False):
    wo_t = W_out.T
    bo = b_out.reshape(1, EMBED)
    out = pl.pallas_call(
        _b_body,
        grid=(NBLK,),
        in_specs=[
            pl.BlockSpec((HEADS, QB, HD), lambda i: (0, i, 0)),
            pl.BlockSpec((HEADS, QB, HD), lambda i: (1, i, 0)),
            pl.BlockSpec((EMBED, EMBED), lambda i: (0, 0)),
            pl.BlockSpec((1, EMBED), lambda i: (0, 0)),
            pl.BlockSpec((QB, EMBED), lambda i: (i, 0)),
        ],
        out_specs=pl.BlockSpec((QB, EMBED), lambda i: (i, 0)),
        out_shape=jax.ShapeDtypeStruct((NQP, EMBED), jnp.float32),
        interpret=interpret,
    )(out_sc, out_sc, wo_t, bo, q_pad)
    return out


def kernel(query, value, reference_points, spatial_shapes, level_start_index,
           W_samp, b_samp, W_attn, b_attn, W_val, b_val, W_out, b_out):
    query2d = query[0]
    value2d = value[0]
    refs2d = reference_points[0, :, 0, :]

    table, idx_all, w_all, q_pad = _tc_prepare(
        query2d, value2d, refs2d, W_samp, b_samp, W_attn, b_attn, W_val, b_val)

    table2d = table.reshape(B2H * NQP, HD)
    idx2d = idx_all.reshape(B2H * NQP * R // GB, GB)
    out_sc = _sc_sample_fn()(table2d, idx2d, w_all)

    out_sc = out_sc.reshape(B2H, NQP, HD)
    out = _tc_finish(out_sc, q_pad, W_out, b_out)
    return out[:NQ].reshape(1, NQ, EMBED)


# speedup vs baseline: 2332.9304x; 1.3646x over previous
import functools

import jax
import jax.numpy as jnp
from jax import lax
from jax.experimental import pallas as pl
from jax.experimental.pallas import tpu as pltpu
from jax.experimental.pallas import tpu_sc as plsc

H = 150
W = 150
NQ = H * W
EMBED = 256
HEADS = 8
POINTS = 4
QUEUE = 2
HD = EMBED // HEADS
B2H = QUEUE * HEADS
R = POINTS * 4

QB = 512
NQP = 44 * QB
NBLK = NQP // QB

NC = 2
NS = 16
LANES = 16
NW = NC * NS

CQ = 128
CHUNKS_PER_B2H = NQP // CQ
CHUNKS_PER_W = CHUNKS_PER_B2H // 2


def _a1_body(v_ref, wv_ref, bv_ref, tab_ref):
    y = jnp.dot(v_ref[...], wv_ref[...], preferred_element_type=jnp.float32)
    y = y + bv_ref[...]
    for h in range(HEADS):
        tab_ref[h] = y[:, h * HD:(h + 1) * HD]


def _a2_body(v0_ref, q_ref, ref_ref, wsx_ref, wsy_ref, bsx_ref, bsy_ref,
             wa_ref, ba_ref, idx_ref, w_ref):
    qe = jnp.concatenate([v0_ref[...], q_ref[...]], axis=1)
    dn = (((1,), (1,)), ((), ()))
    sox = lax.dot_general(wsx_ref[...], qe, dn,
                          preferred_element_type=jnp.float32) + bsx_ref[...]
    soy = lax.dot_general(wsy_ref[...], qe, dn,
                          preferred_element_type=jnp.float32) + bsy_ref[...]
    a = lax.dot_general(wa_ref[...], qe, dn,
                        preferred_element_type=jnp.float32) + ba_ref[...]
    s = jnp.exp(a)
    gi = lax.broadcasted_iota(jnp.int32, (64, 64), 0) // POINTS
    gj = lax.broadcasted_iota(jnp.int32, (64, 64), 1) // POINTS
    gmat = (gi == gj).astype(jnp.float32)
    denom = jnp.dot(gmat, s, preferred_element_type=jnp.float32)
    attn = s / denom

    refx = ref_ref[0:1, :]
    refy = ref_ref[1:2, :]
    x = refx * W + sox - 0.5
    y = refy * H + soy - 0.5
    x0 = jnp.floor(x)
    y0 = jnp.floor(y)
    fx = x - x0
    fy = y - y0
    x0i = x0.astype(jnp.int32)
    y0i = y0.astype(jnp.int32)

    row = lax.broadcasted_iota(jnp.int32, (64, QB), 0)
    b2h_off = (row // POINTS) * NQP

    idx_parts = []
    w_parts = []
    for dx, dy, wc in ((0, 0, (1.0 - fx) * (1.0 - fy)),
                       (1, 0, fx * (1.0 - fy)),
                       (0, 1, (1.0 - fx) * fy),
                       (1, 1, fx * fy)):
        xi = x0i + dx
        yi = y0i + dy
        valid = (xi >= 0) & (xi <= W - 1) & (yi >= 0) & (yi <= H - 1)
        xc = jnp.clip(xi, 0, W - 1)
        yc = jnp.clip(yi, 0, H - 1)
        idx_parts.append(b2h_off + yc * W + xc)
        w_parts.append(wc * attn * valid.astype(jnp.float32))
    for b in range(B2H):
        sl = slice(b * POINTS, (b + 1) * POINTS)
        idx_ref[b] = jnp.concatenate([p_[sl, :] for p_ in idx_parts], axis=0)
        w_ref[b] = jnp.concatenate([p_[sl, :] for p_ in w_parts], axis=0)


def _b_body(s0_ref, s1_ref, wo_ref, bo_ref, q_ref, out_ref):
    a = jnp.concatenate([s0_ref[h] for h in range(HEADS)], axis=1)
    b = jnp.concatenate([s1_ref[h] for h in range(HEADS)], axis=1)
    m = (a + b) * 0.5
    out_ref[...] = (jnp.dot(m, wo_ref[...], preferred_element_type=jnp.float32)
                    + bo_ref[...] + q_ref[...])


def _tc_prepare(query2d, value2d, refs2d, W_samp, b_samp, W_attn, b_attn,
                W_val, b_val, *, interpret=False):
    padq = NQP - NQ
    q_pad = jnp.pad(query2d, ((0, padq), (0, 0)))
    v3 = value2d.reshape(QUEUE, NQ, EMBED)
    v_pad = jnp.pad(v3, ((0, 0), (0, padq), (0, 0))).reshape(QUEUE * NQP, EMBED)
    refs_pad = jnp.pad(refs2d, ((0, padq), (0, 0)))

    wv_t = W_val.T
    bv = b_val.reshape(1, EMBED)
    table = pl.pallas_call(
        _a1_body,
        grid=(QUEUE, NBLK),
        in_specs=[
            pl.BlockSpec((QB, EMBED), lambda qq, i: (qq * NBLK + i, 0)),
            pl.BlockSpec((EMBED, EMBED), lambda qq, i: (0, 0)),
            pl.BlockSpec((1, EMBED), lambda qq, i: (0, 0)),
        ],
        out_specs=pl.BlockSpec((HEADS, QB, HD), lambda qq, i: (qq, i, 0)),
        out_shape=jax.ShapeDtypeStruct((B2H, NQP, HD), jnp.float32),
        interpret=interpret,
    )(v_pad, wv_t, bv)

    cols = jnp.arange(64, dtype=jnp.int32)
    qq_c = cols // 32
    h_c = (cols // POINTS) % HEADS
    p_c = cols % POINTS
    perm = (h_c * QUEUE + qq_c) * POINTS + p_c
    wsx = W_samp[2 * perm, :]
    wsy = W_samp[2 * perm + 1, :]
    bsx = b_samp[2 * perm].reshape(64, 1)
    bsy = b_samp[2 * perm + 1].reshape(64, 1)
    wa = W_attn[perm, :]
    ba = b_attn[perm].reshape(64, 1)
    refs_t = refs_pad.T

    idx_nat, w_nat = pl.pallas_call(
        _a2_body,
        grid=(NBLK,),
        in_specs=[
            pl.BlockSpec((QB, EMBED), lambda i: (i, 0)),
            pl.BlockSpec((QB, EMBED), lambda i: (i, 0)),
            pl.BlockSpec((2, QB), lambda i: (0, i)),
            pl.BlockSpec((64, 2 * EMBED), lambda i: (0, 0)),
            pl.BlockSpec((64, 2 * EMBED), lambda i: (0, 0)),
            pl.BlockSpec((64, 1), lambda i: (0, 0)),
            pl.BlockSpec((64, 1), lambda i: (0, 0)),
            pl.BlockSpec((64, 2 * EMBED), lambda i: (0, 0)),
            pl.BlockSpec((64, 1), lambda i: (0, 0)),
        ],
        out_specs=[
            pl.BlockSpec((B2H, R, QB), lambda i: (0, 0, i)),
            pl.BlockSpec((B2H, R, QB), lambda i: (0, 0, i)),
        ],
        out_shape=[
            jax.ShapeDtypeStruct((B2H, R, NQP), jnp.int32),
            jax.ShapeDtypeStruct((B2H, R, NQP), jnp.float32),
        ],
        interpret=interpret,
    )(v_pad[:NQP], q_pad, refs_t, wsx, wsy, bsx, bsy, wa, ba)

    idx2d = idx_nat.reshape(B2H * R, NQP)
    w2d = w_nat.reshape(B2H * R, NQP)
    return table, idx2d, w2d, q_pad


def _bcast_lane(vec, r):
    idx = jnp.full((LANES, 1), r, jnp.int32)
    dn = lax.GatherDimensionNumbers(offset_dims=(), collapsed_slice_dims=(0,),
                                    start_index_map=(0,))
    return lax.gather(vec, idx, dn, (1,),
                      mode=lax.GatherScatterMode.PROMISE_IN_BOUNDS)


def _sc_body(table_hbm, idx_hbm, w_hbm, out_hbm, idx_v, w_v, rows_v, out_v, sem):
    cid = lax.axis_index("c")
    sid = lax.axis_index("s")
    wid = sid * NC + cid
    b2h = wid // 2
    half = wid % 2
    iota = lax.iota(jnp.int32, LANES)

    def chunk_body(i, carry):
        qpos = (half * CHUNKS_PER_W + i) * CQ
        pltpu.sync_copy(idx_hbm.at[pl.ds(b2h * R, R), pl.ds(qpos, CQ)], idx_v)
        pltpu.sync_copy(w_hbm.at[pl.ds(b2h * R, R), pl.ds(qpos, CQ)], w_v)
        copies = [
            pltpu.async_copy(table_hbm.at[idx_v.at[r]],
                             rows_v.at[pl.ds(r * CQ, CQ)], sem)
            for r in range(R)
        ]
        for c in copies:
            c.wait()

        def q_body(j, carry2):
            wv = plsc.load_gather(w_v, [iota, jnp.full((LANES,), 0, jnp.int32) + j])
            acc0 = jnp.zeros((LANES,), jnp.float32)
            acc1 = jnp.zeros((LANES,), jnp.float32)
            for r in range(R):
                wb = _bcast_lane(wv, r)
                jr = r * CQ + j
                acc0 = acc0 + rows_v[jr, pl.ds(0, LANES)] * wb
                acc1 = acc1 + rows_v[jr, pl.ds(LANES, LANES)] * wb
            out_v[pl.ds(j * HD, LANES)] = acc0
            out_v[pl.ds(j * HD + LANES, LANES)] = acc1
            return carry2

        lax.fori_loop(0, CQ, q_body, 0)
        pltpu.sync_copy(out_v,
                        out_hbm.at[pl.ds(b2h * (NQP * HD) + qpos * HD, CQ * HD)])
        return carry

    lax.fori_loop(0, CHUNKS_PER_W, chunk_body, 0)


@functools.lru_cache(maxsize=1)
def _sc_sample_fn():
    mesh = plsc.VectorSubcoreMesh(core_axis_name="c", subcore_axis_name="s")
    return pl.kernel(
        _sc_body,
        out_type=jax.ShapeDtypeStruct((B2H * NQP * HD,), jnp.float32),
        mesh=mesh,
        scratch_types=[
            pltpu.VMEM((R, CQ), jnp.int32),
            pltpu.VMEM((R, CQ), jnp.float32),
            pltpu.VMEM((R * CQ, HD), jnp.float32),
            pltpu.VMEM((CQ * HD,), jnp.float32),
            pltpu.SemaphoreType.DMA,
        ],
        compiler_params=pltpu.CompilerParams(use_tc_tiling_on_sc=False,
                                             needs_layout_passes=False),
    )


def _tc_finish(out_sc, q_pad, W_out, b_out, *, interpret=False):
    wo_t = W_out.T
    bo = b_out.reshape(1, EMBED)
    out = pl.pallas_call(
        _b_body,
        grid=(NBLK,),
        in_specs=[
            pl.BlockSpec((HEADS, QB, HD), lambda i: (0, i, 0)),
            pl.BlockSpec((HEADS, QB, HD), lambda i: (1, i, 0)),
            pl.BlockSpec((EMBED, EMBED), lambda i: (0, 0)),
            pl.BlockSpec((1, EMBED), lambda i: (0, 0)),
            pl.BlockSpec((QB, EMBED), lambda i: (i, 0)),
        ],
        out_specs=pl.BlockSpec((QB, EMBED), lambda i: (i, 0)),
        out_shape=jax.ShapeDtypeStruct((NQP, EMBED), jnp.float32),
        interpret=interpret,
    )(out_sc, out_sc, wo_t, bo, q_pad)
    return out


def kernel(query, value, reference_points, spatial_shapes, level_start_index,
           W_samp, b_samp, W_attn, b_attn, W_val, b_val, W_out, b_out):
    query2d = query[0]
    value2d = value[0]
    refs2d = reference_points[0, :, 0, :]

    table, idx2d, w2d, q_pad = _tc_prepare(
        query2d, value2d, refs2d, W_samp, b_samp, W_attn, b_attn, W_val, b_val)

    table2d = table.reshape(B2H * NQP, HD)
    out_sc = _sc_sample_fn()(table2d, idx2d, w2d)

    out_sc = out_sc.reshape(B2H, NQP, HD)
    out = _tc_finish(out_sc, q_pad, W_out, b_out)
    return out[:NQ].reshape(1, NQ, EMBED)


# speedup vs baseline: 2609.8161x; 1.1187x over previous
import functools

import jax
import jax.numpy as jnp
from jax import lax
from jax.experimental import pallas as pl
from jax.experimental.pallas import tpu as pltpu
from jax.experimental.pallas import tpu_sc as plsc

H = 150
W = 150
NQ = H * W
EMBED = 256
HEADS = 8
POINTS = 4
QUEUE = 2
HD = EMBED // HEADS
B2H = QUEUE * HEADS
R = POINTS * 4

QB = 512
NQP = 44 * QB
NBLK = NQP // QB

NC = 2
NS = 16
LANES = 16
NW = NC * NS

CQ = 64
CHUNKS_PER_B2H = NQP // CQ
CHUNKS_PER_W = CHUNKS_PER_B2H // 2


def _a1_body(v_ref, wv_ref, bv_ref, tab_ref):
    y = jnp.dot(v_ref[...], wv_ref[...], preferred_element_type=jnp.float32)
    y = y + bv_ref[...]
    for h in range(HEADS):
        tab_ref[h] = y[:, h * HD:(h + 1) * HD]


def _a2_body(v0_ref, q_ref, ref_ref, wsx_ref, wsy_ref, bsx_ref, bsy_ref,
             wa_ref, ba_ref, idx_ref, w_ref):
    qe = jnp.concatenate([v0_ref[...], q_ref[...]], axis=1)
    dn = (((1,), (1,)), ((), ()))
    sox = lax.dot_general(wsx_ref[...], qe, dn,
                          preferred_element_type=jnp.float32) + bsx_ref[...]
    soy = lax.dot_general(wsy_ref[...], qe, dn,
                          preferred_element_type=jnp.float32) + bsy_ref[...]
    a = lax.dot_general(wa_ref[...], qe, dn,
                        preferred_element_type=jnp.float32) + ba_ref[...]
    s = jnp.exp(a)
    gi = lax.broadcasted_iota(jnp.int32, (64, 64), 0) // POINTS
    gj = lax.broadcasted_iota(jnp.int32, (64, 64), 1) // POINTS
    gmat = (gi == gj).astype(jnp.float32)
    denom = jnp.dot(gmat, s, preferred_element_type=jnp.float32)
    attn = s / denom

    refx = ref_ref[0:1, :]
    refy = ref_ref[1:2, :]
    x = refx * W + sox - 0.5
    y = refy * H + soy - 0.5
    x0 = jnp.floor(x)
    y0 = jnp.floor(y)
    fx = x - x0
    fy = y - y0
    x0i = x0.astype(jnp.int32)
    y0i = y0.astype(jnp.int32)

    row = lax.broadcasted_iota(jnp.int32, (64, QB), 0)
    b2h_off = (row // POINTS) * NQP

    idx_parts = []
    w_parts = []
    for dx, dy, wc in ((0, 0, (1.0 - fx) * (1.0 - fy)),
                       (1, 0, fx * (1.0 - fy)),
                       (0, 1, (1.0 - fx) * fy),
                       (1, 1, fx * fy)):
        xi = x0i + dx
        yi = y0i + dy
        valid = (xi >= 0) & (xi <= W - 1) & (yi >= 0) & (yi <= H - 1)
        xc = jnp.clip(xi, 0, W - 1)
        yc = jnp.clip(yi, 0, H - 1)
        idx_parts.append(b2h_off + yc * W + xc)
        w_parts.append(wc * attn * valid.astype(jnp.float32))
    for b in range(B2H):
        sl = slice(b * POINTS, (b + 1) * POINTS)
        idx_ref[b] = jnp.concatenate([p_[sl, :] for p_ in idx_parts], axis=0)
        w_ref[b] = jnp.concatenate([p_[sl, :] for p_ in w_parts], axis=0)


def _b_body(s0_ref, s1_ref, wo_ref, bo_ref, q_ref, out_ref):
    a = jnp.concatenate([s0_ref[h] for h in range(HEADS)], axis=1)
    b = jnp.concatenate([s1_ref[h] for h in range(HEADS)], axis=1)
    m = (a + b) * 0.5
    out_ref[...] = (jnp.dot(m, wo_ref[...], preferred_element_type=jnp.float32)
                    + bo_ref[...] + q_ref[...])


def _tc_prepare(query2d, value2d, refs2d, W_samp, b_samp, W_attn, b_attn,
                W_val, b_val, *, interpret=False):
    padq = NQP - NQ
    q_pad = jnp.pad(query2d, ((0, padq), (0, 0)))
    v3 = value2d.reshape(QUEUE, NQ, EMBED)
    v_pad = jnp.pad(v3, ((0, 0), (0, padq), (0, 0))).reshape(QUEUE * NQP, EMBED)
    refs_pad = jnp.pad(refs2d, ((0, padq), (0, 0)))

    wv_t = W_val.T
    bv = b_val.reshape(1, EMBED)
    table = pl.pallas_call(
        _a1_body,
        grid=(QUEUE, NBLK),
        in_specs=[
            pl.BlockSpec((QB, EMBED), lambda qq, i: (qq * NBLK + i, 0)),
            pl.BlockSpec((EMBED, EMBED), lambda qq, i: (0, 0)),
            pl.BlockSpec((1, EMBED), lambda qq, i: (0, 0)),
        ],
        out_specs=pl.BlockSpec((HEADS, QB, HD), lambda qq, i: (qq, i, 0)),
        out_shape=jax.ShapeDtypeStruct((B2H, NQP, HD), jnp.float32),
        interpret=interpret,
    )(v_pad, wv_t, bv)

    cols = jnp.arange(64, dtype=jnp.int32)
    qq_c = cols // 32
    h_c = (cols // POINTS) % HEADS
    p_c = cols % POINTS
    perm = (h_c * QUEUE + qq_c) * POINTS + p_c
    wsx = W_samp[2 * perm, :]
    wsy = W_samp[2 * perm + 1, :]
    bsx = b_samp[2 * perm].reshape(64, 1)
    bsy = b_samp[2 * perm + 1].reshape(64, 1)
    wa = W_attn[perm, :]
    ba = b_attn[perm].reshape(64, 1)
    refs_t = refs_pad.T

    idx_nat, w_nat = pl.pallas_call(
        _a2_body,
        grid=(NBLK,),
        in_specs=[
            pl.BlockSpec((QB, EMBED), lambda i: (i, 0)),
            pl.BlockSpec((QB, EMBED), lambda i: (i, 0)),
            pl.BlockSpec((2, QB), lambda i: (0, i)),
            pl.BlockSpec((64, 2 * EMBED), lambda i: (0, 0)),
            pl.BlockSpec((64, 2 * EMBED), lambda i: (0, 0)),
            pl.BlockSpec((64, 1), lambda i: (0, 0)),
            pl.BlockSpec((64, 1), lambda i: (0, 0)),
            pl.BlockSpec((64, 2 * EMBED), lambda i: (0, 0)),
            pl.BlockSpec((64, 1), lambda i: (0, 0)),
        ],
        out_specs=[
            pl.BlockSpec((B2H, R, QB), lambda i: (0, 0, i)),
            pl.BlockSpec((B2H, R, QB), lambda i: (0, 0, i)),
        ],
        out_shape=[
            jax.ShapeDtypeStruct((B2H, R, NQP), jnp.int32),
            jax.ShapeDtypeStruct((B2H, R, NQP), jnp.float32),
        ],
        interpret=interpret,
    )(v_pad[:NQP], q_pad, refs_t, wsx, wsy, bsx, bsy, wa, ba)

    idx2d = idx_nat.reshape(B2H * R, NQP)
    w2d = w_nat.reshape(B2H * R, NQP)
    return table, idx2d, w2d, q_pad


def _bcast_lane(vec, r):
    idx = jnp.full((LANES, 1), r, jnp.int32)
    dn = lax.GatherDimensionNumbers(offset_dims=(), collapsed_slice_dims=(0,),
                                    start_index_map=(0,))
    return lax.gather(vec, idx, dn, (1,),
                      mode=lax.GatherScatterMode.PROMISE_IN_BOUNDS)


def _sc_body(table_hbm, idx_hbm, w_hbm, out_hbm, idx_v, w_v, rows_v, out_v,
             sem0, sem1):
    cid = lax.axis_index("c")
    sid = lax.axis_index("s")
    wid = sid * NC + cid
    b2h = wid // 2
    half = wid % 2
    iota = lax.iota(jnp.int32, LANES)
    nlc = CHUNKS_PER_W

    def fetch_idx(c, buf):
        qpos = (half * nlc + c) * CQ
        pltpu.sync_copy(idx_hbm.at[pl.ds(b2h * R, R), pl.ds(qpos, CQ)],
                        idx_v.at[buf])
        pltpu.sync_copy(w_hbm.at[pl.ds(b2h * R, R), pl.ds(qpos, CQ)],
                        w_v.at[buf])

    def gather_copies(buf):
        sem = sem0 if buf == 0 else sem1
        return [
            pltpu.make_async_copy(table_hbm.at[idx_v.at[buf, r]],
                                  rows_v.at[buf, pl.ds(r * CQ, CQ)], sem)
            for r in range(R)
        ]

    fetch_idx(0, 0)
    for cp in gather_copies(0):
        cp.start()
    fetch_idx(jnp.minimum(1, nlc - 1), 1)

    def pair_body(k, carry):
        for par in (0, 1):
            c = 2 * k + par
            buf = par
            nbuf = 1 - par
            for cp in gather_copies(buf):
                cp.wait()
            for cp in gather_copies(nbuf):
                cp.start()
            rv = rows_v.at[buf]
            wvr = w_v.at[buf]

            def q_body(j, carry2):
                wv = plsc.load_gather(wvr, [iota, jnp.zeros((LANES,), jnp.int32) + j])
                acc0 = jnp.zeros((LANES,), jnp.float32)
                acc1 = jnp.zeros((LANES,), jnp.float32)
                for r in range(R):
                    wb = _bcast_lane(wv, r)
                    jr = r * CQ + j
                    acc0 = acc0 + rv[jr, pl.ds(0, LANES)] * wb
                    acc1 = acc1 + rv[jr, pl.ds(LANES, LANES)] * wb
                out_v[pl.ds(j * HD, LANES)] = acc0
                out_v[pl.ds(j * HD + LANES, LANES)] = acc1
                return carry2

            lax.fori_loop(0, CQ, q_body, 0)
            qpos = (half * nlc + c) * CQ
            pltpu.sync_copy(out_v,
                            out_hbm.at[pl.ds(b2h * (NQP * HD) + qpos * HD,
                                             CQ * HD)])
            fetch_idx(jnp.minimum(c + 2, nlc - 1), buf)
        return carry

    lax.fori_loop(0, nlc // 2, pair_body, 0)
    for cp in gather_copies(0):
        cp.wait()


@functools.lru_cache(maxsize=1)
def _sc_sample_fn():
    mesh = plsc.VectorSubcoreMesh(core_axis_name="c", subcore_axis_name="s")
    return pl.kernel(
        _sc_body,
        out_type=jax.ShapeDtypeStruct((B2H * NQP * HD,), jnp.float32),
        mesh=mesh,
        scratch_types=[
            pltpu.VMEM((2, R, CQ), jnp.int32),
            pltpu.VMEM((2, R, CQ), jnp.float32),
            pltpu.VMEM((2, R * CQ, HD), jnp.float32),
            pltpu.VMEM((CQ * HD,), jnp.float32),
            pltpu.SemaphoreType.DMA,
            pltpu.SemaphoreType.DMA,
        ],
        compiler_params=pltpu.CompilerParams(use_tc_tiling_on_sc=False,
                                             needs_layout_passes=False),
    )


def _tc_finish(out_sc, q_pad, W_out, b_out, *, interpret=False):
    wo_t = W_out.T
    bo = b_out.reshape(1, EMBED)
    out = pl.pallas_call(
        _b_body,
        grid=(NBLK,),
        in_specs=[
            pl.BlockSpec((HEADS, QB, HD), lambda i: (0, i, 0)),
            pl.BlockSpec((HEADS, QB, HD), lambda i: (1, i, 0)),
            pl.BlockSpec((EMBED, EMBED), lambda i: (0, 0)),
            pl.BlockSpec((1, EMBED), lambda i: (0, 0)),
            pl.BlockSpec((QB, EMBED), lambda i: (i, 0)),
        ],
        out_specs=pl.BlockSpec((QB, EMBED), lambda i: (i, 0)),
        out_shape=jax.ShapeDtypeStruct((NQP, EMBED), jnp.float32),
        interpret=interpret,
    )(out_sc, out_sc, wo_t, bo, q_pad)
    return out


def kernel(query, value, reference_points, spatial_shapes, level_start_index,
           W_samp, b_samp, W_attn, b_attn, W_val, b_val, W_out, b_out):
    query2d = query[0]
    value2d = value[0]
    refs2d = reference_points[0, :, 0, :]

    table, idx2d, w2d, q_pad = _tc_prepare(
        query2d, value2d, refs2d, W_samp, b_samp, W_attn, b_attn, W_val, b_val)

    table2d = table.reshape(B2H * NQP, HD)
    out_sc = _sc_sample_fn()(table2d, idx2d, w2d)

    out_sc = out_sc.reshape(B2H, NQP, HD)
    out = _tc_finish(out_sc, q_pad, W_out, b_out)
    return out[:NQ].reshape(1, NQ, EMBED)


# speedup vs baseline: 2819.3039x; 1.0803x over previous
import functools

import jax
import jax.numpy as jnp
from jax import lax
from jax.experimental import pallas as pl
from jax.experimental.pallas import tpu as pltpu
from jax.experimental.pallas import tpu_sc as plsc

H = 150
W = 150
NQ = H * W
EMBED = 256
HEADS = 8
POINTS = 4
QUEUE = 2
HD = EMBED // HEADS
B2H = QUEUE * HEADS
R = POINTS * 4

QB = 512
NQP = 44 * QB
NBLK = NQP // QB

NC = 2
NS = 16
LANES = 16
NW = NC * NS

CQ = 64
CHUNKS_PER_B2H = NQP // CQ
CHUNKS_PER_W = CHUNKS_PER_B2H // 2


def _a1_body(v_ref, wv_ref, bv_ref, tab_ref):
    y = jnp.dot(v_ref[0], wv_ref[...], preferred_element_type=jnp.float32)
    y = y + bv_ref[...]
    for h in range(HEADS):
        tab_ref[h] = y[:, h * HD:(h + 1) * HD]


def _a2_body(v0_ref, q_ref, ref_ref, wsx_ref, wsy_ref, bsx_ref, bsy_ref,
             wa_ref, ba_ref, idx_ref, w_ref):
    qe = jnp.concatenate([v0_ref[0], q_ref[...]], axis=1)
    dn = (((1,), (1,)), ((), ()))
    sox = lax.dot_general(wsx_ref[...], qe, dn,
                          preferred_element_type=jnp.float32) + bsx_ref[...]
    soy = lax.dot_general(wsy_ref[...], qe, dn,
                          preferred_element_type=jnp.float32) + bsy_ref[...]
    a = lax.dot_general(wa_ref[...], qe, dn,
                        preferred_element_type=jnp.float32) + ba_ref[...]
    s = jnp.exp(a)
    gi = lax.broadcasted_iota(jnp.int32, (64, 64), 0) // POINTS
    gj = lax.broadcasted_iota(jnp.int32, (64, 64), 1) // POINTS
    gmat = (gi == gj).astype(jnp.float32)
    denom = jnp.dot(gmat, s, preferred_element_type=jnp.float32)
    attn = s / denom

    refx = ref_ref[0:1, :]
    refy = ref_ref[1:2, :]
    x = refx * W + sox - 0.5
    y = refy * H + soy - 0.5
    x0 = jnp.floor(x)
    y0 = jnp.floor(y)
    fx = x - x0
    fy = y - y0
    x0i = x0.astype(jnp.int32)
    y0i = y0.astype(jnp.int32)

    row = lax.broadcasted_iota(jnp.int32, (64, QB), 0)
    b2h_off = (row // POINTS) * NQP

    idx_parts = []
    w_parts = []
    for dx, dy, wc in ((0, 0, (1.0 - fx) * (1.0 - fy)),
                       (1, 0, fx * (1.0 - fy)),
                       (0, 1, (1.0 - fx) * fy),
                       (1, 1, fx * fy)):
        xi = x0i + dx
        yi = y0i + dy
        valid = (xi >= 0) & (xi <= W - 1) & (yi >= 0) & (yi <= H - 1)
        xc = jnp.clip(xi, 0, W - 1)
        yc = jnp.clip(yi, 0, H - 1)
        idx_parts.append(b2h_off + yc * W + xc)
        w_parts.append(wc * attn * valid.astype(jnp.float32))
    for b in range(B2H):
        sl = slice(b * POINTS, (b + 1) * POINTS)
        idx_ref[b] = jnp.concatenate([p_[sl, :] for p_ in idx_parts], axis=0)
        w_ref[b] = jnp.concatenate([p_[sl, :] for p_ in w_parts], axis=0)


def _b_body(s0_ref, s1_ref, wo_ref, bo_ref, q_ref, out_ref):
    a = jnp.concatenate([s0_ref[h] for h in range(HEADS)], axis=1)
    b = jnp.concatenate([s1_ref[h] for h in range(HEADS)], axis=1)
    m = (a + b) * 0.5
    out_ref[...] = (jnp.dot(m, wo_ref[...], preferred_element_type=jnp.float32)
                    + bo_ref[...] + q_ref[...])


def _tc_prepare(query2d, value2d, refs2d, W_samp, b_samp, W_attn, b_attn,
                W_val, b_val, *, interpret=False):
    v3 = value2d.reshape(QUEUE, NQ, EMBED)

    wv_t = W_val.T
    bv = b_val.reshape(1, EMBED)
    table = pl.pallas_call(
        _a1_body,
        grid=(QUEUE, NBLK),
        in_specs=[
            pl.BlockSpec((1, QB, EMBED), lambda qq, i: (qq, i, 0)),
            pl.BlockSpec((EMBED, EMBED), lambda qq, i: (0, 0)),
            pl.BlockSpec((1, EMBED), lambda qq, i: (0, 0)),
        ],
        out_specs=pl.BlockSpec((HEADS, QB, HD), lambda qq, i: (qq, i, 0)),
        out_shape=jax.ShapeDtypeStruct((B2H, NQP, HD), jnp.float32),
        interpret=interpret,
    )(v3, wv_t, bv)

    cols = jnp.arange(64, dtype=jnp.int32)
    qq_c = cols // 32
    h_c = (cols // POINTS) % HEADS
    p_c = cols % POINTS
    perm = (h_c * QUEUE + qq_c) * POINTS + p_c
    wsx = W_samp[2 * perm, :]
    wsy = W_samp[2 * perm + 1, :]
    bsx = b_samp[2 * perm].reshape(64, 1)
    bsy = b_samp[2 * perm + 1].reshape(64, 1)
    wa = W_attn[perm, :]
    ba = b_attn[perm].reshape(64, 1)
    refs_t = refs2d.T

    idx_nat, w_nat = pl.pallas_call(
        _a2_body,
        grid=(NBLK,),
        in_specs=[
            pl.BlockSpec((1, QB, EMBED), lambda i: (0, i, 0)),
            pl.BlockSpec((QB, EMBED), lambda i: (i, 0)),
            pl.BlockSpec((2, QB), lambda i: (0, i)),
            pl.BlockSpec((64, 2 * EMBED), lambda i: (0, 0)),
            pl.BlockSpec((64, 2 * EMBED), lambda i: (0, 0)),
            pl.BlockSpec((64, 1), lambda i: (0, 0)),
            pl.BlockSpec((64, 1), lambda i: (0, 0)),
            pl.BlockSpec((64, 2 * EMBED), lambda i: (0, 0)),
            pl.BlockSpec((64, 1), lambda i: (0, 0)),
        ],
        out_specs=[
            pl.BlockSpec((B2H, R, QB), lambda i: (0, 0, i)),
            pl.BlockSpec((B2H, R, QB), lambda i: (0, 0, i)),
        ],
        out_shape=[
            jax.ShapeDtypeStruct((B2H, R, NQP), jnp.int32),
            jax.ShapeDtypeStruct((B2H, R, NQP), jnp.float32),
        ],
        interpret=interpret,
    )(v3, query2d, refs_t, wsx, wsy, bsx, bsy, wa, ba)

    idx2d = idx_nat.reshape(B2H * R, NQP)
    w2d = w_nat.reshape(B2H * R, NQP)
    return table, idx2d, w2d


def _bcast_lane(vec, r):
    idx = jnp.full((LANES, 1), r, jnp.int32)
    dn = lax.GatherDimensionNumbers(offset_dims=(), collapsed_slice_dims=(0,),
                                    start_index_map=(0,))
    return lax.gather(vec, idx, dn, (1,),
                      mode=lax.GatherScatterMode.PROMISE_IN_BOUNDS)


def _sc_body(table_hbm, idx_hbm, w_hbm, out_hbm, idx_v, w_v, rows_v, out_v,
             sem0, sem1):
    cid = lax.axis_index("c")
    sid = lax.axis_index("s")
    wid = sid * NC + cid
    b2h = wid // 2
    half = wid % 2
    iota = lax.iota(jnp.int32, LANES)
    nlc = CHUNKS_PER_W

    def fetch_idx(c, buf):
        qpos = (half * nlc + c) * CQ
        pltpu.sync_copy(idx_hbm.at[pl.ds(b2h * R, R), pl.ds(qpos, CQ)],
                        idx_v.at[buf])
        pltpu.sync_copy(w_hbm.at[pl.ds(b2h * R, R), pl.ds(qpos, CQ)],
                        w_v.at[buf])

    def gather_copies(buf):
        sem = sem0 if buf == 0 else sem1
        return [
            pltpu.make_async_copy(table_hbm.at[idx_v.at[buf, r]],
                                  rows_v.at[buf, pl.ds(r * CQ, CQ)], sem)
            for r in range(R)
        ]

    fetch_idx(0, 0)
    for cp in gather_copies(0):
        cp.start()
    fetch_idx(jnp.minimum(1, nlc - 1), 1)

    def pair_body(k, carry):
        for par in (0, 1):
            c = 2 * k + par
            buf = par
            nbuf = 1 - par
            for cp in gather_copies(buf):
                cp.wait()
            for cp in gather_copies(nbuf):
                cp.start()
            rv = rows_v.at[buf]
            wvr = w_v.at[buf]

            def q_body(j, carry2):
                wv = plsc.load_gather(wvr, [iota, jnp.zeros((LANES,), jnp.int32) + j])
                acc0 = jnp.zeros((LANES,), jnp.float32)
                acc1 = jnp.zeros((LANES,), jnp.float32)
                for r in range(R):
                    wb = _bcast_lane(wv, r)
                    jr = r * CQ + j
                    acc0 = acc0 + rv[jr, pl.ds(0, LANES)] * wb
                    acc1 = acc1 + rv[jr, pl.ds(LANES, LANES)] * wb
                out_v[pl.ds(j * HD, LANES)] = acc0
                out_v[pl.ds(j * HD + LANES, LANES)] = acc1
                return carry2

            lax.fori_loop(0, CQ, q_body, 0)
            qpos = (half * nlc + c) * CQ
            pltpu.sync_copy(out_v,
                            out_hbm.at[pl.ds(b2h * (NQP * HD) + qpos * HD,
                                             CQ * HD)])
            fetch_idx(jnp.minimum(c + 2, nlc - 1), buf)
        return carry

    lax.fori_loop(0, nlc // 2, pair_body, 0)
    for cp in gather_copies(0):
        cp.wait()


@functools.lru_cache(maxsize=1)
def _sc_sample_fn():
    mesh = plsc.VectorSubcoreMesh(core_axis_name="c", subcore_axis_name="s")
    return pl.kernel(
        _sc_body,
        out_type=jax.ShapeDtypeStruct((B2H * NQP * HD,), jnp.float32),
        mesh=mesh,
        scratch_types=[
            pltpu.VMEM((2, R, CQ), jnp.int32),
            pltpu.VMEM((2, R, CQ), jnp.float32),
            pltpu.VMEM((2, R * CQ, HD), jnp.float32),
            pltpu.VMEM((CQ * HD,), jnp.float32),
            pltpu.SemaphoreType.DMA,
            pltpu.SemaphoreType.DMA,
        ],
        compiler_params=pltpu.CompilerParams(use_tc_tiling_on_sc=False,
                                             needs_layout_passes=False),
    )


def _tc_finish(out_sc, query2d, W_out, b_out, *, interpret=False):
    wo_t = W_out.T
    bo = b_out.reshape(1, EMBED)
    out = pl.pallas_call(
        _b_body,
        grid=(NBLK,),
        in_specs=[
            pl.BlockSpec((HEADS, QB, HD), lambda i: (0, i, 0)),
            pl.BlockSpec((HEADS, QB, HD), lambda i: (1, i, 0)),
            pl.BlockSpec((EMBED, EMBED), lambda i: (0, 0)),
            pl.BlockSpec((1, EMBED), lambda i: (0, 0)),
            pl.BlockSpec((QB, EMBED), lambda i: (i, 0)),
        ],
        out_specs=pl.BlockSpec((QB, EMBED), lambda i: (i, 0)),
        out_shape=jax.ShapeDtypeStruct((NQ, EMBED), jnp.float32),
        interpret=interpret,
    )(out_sc, out_sc, wo_t, bo, query2d)
    return out


def kernel(query, value, reference_points, spatial_shapes, level_start_index,
           W_samp, b_samp, W_attn, b_attn, W_val, b_val, W_out, b_out):
    query2d = query[0]
    value2d = value[0]
    refs2d = reference_points[0, :, 0, :]

    table, idx2d, w2d = _tc_prepare(
        query2d, value2d, refs2d, W_samp, b_samp, W_attn, b_attn, W_val, b_val)

    table2d = table.reshape(B2H * NQP, HD)
    out_sc = _sc_sample_fn()(table2d, idx2d, w2d)

    out_sc = out_sc.reshape(B2H, NQP, HD)
    out = _tc_finish(out_sc, query2d, W_out, b_out)
    return out.reshape(1, NQ, EMBED)


# speedup vs baseline: 2821.8038x; 1.0009x over previous
import functools

import jax
import jax.numpy as jnp
from jax import lax
from jax.experimental import pallas as pl
from jax.experimental.pallas import tpu as pltpu
from jax.experimental.pallas import tpu_sc as plsc

H = 150
W = 150
NQ = H * W
EMBED = 256
HEADS = 8
POINTS = 4
QUEUE = 2
HD = EMBED // HEADS
B2H = QUEUE * HEADS
R = POINTS * 4

QB = 512
NQP = 44 * QB
NBLK = NQP // QB

NC = 2
NS = 16
LANES = 16
NW = NC * NS

CQ = 64
CHUNKS_PER_B2H = NQP // CQ
CHUNKS_PER_W = CHUNKS_PER_B2H // 2


def _a1_body(v_ref, wv_ref, bv_ref, tab_ref):
    y = jnp.dot(v_ref[0], wv_ref[...], preferred_element_type=jnp.float32)
    y = y + bv_ref[...]
    for h in range(HEADS):
        tab_ref[h] = y[:, h * HD:(h + 1) * HD]


def _a2_body(v0_ref, q_ref, ref_ref, wsx_ref, wsy_ref, bsx_ref, bsy_ref,
             wa_ref, ba_ref, idx_ref, w_ref):
    qe = jnp.concatenate([v0_ref[0], q_ref[...]], axis=1)
    dn = (((1,), (1,)), ((), ()))
    sox = lax.dot_general(wsx_ref[...], qe, dn,
                          preferred_element_type=jnp.float32) + bsx_ref[...]
    soy = lax.dot_general(wsy_ref[...], qe, dn,
                          preferred_element_type=jnp.float32) + bsy_ref[...]
    a = lax.dot_general(wa_ref[...], qe, dn,
                        preferred_element_type=jnp.float32) + ba_ref[...]
    s = jnp.exp(a)
    gi = lax.broadcasted_iota(jnp.int32, (64, 64), 0) // POINTS
    gj = lax.broadcasted_iota(jnp.int32, (64, 64), 1) // POINTS
    gmat = (gi == gj).astype(jnp.float32)
    denom = jnp.dot(gmat, s, preferred_element_type=jnp.float32)
    attn = s / denom

    refx = ref_ref[0:1, :]
    refy = ref_ref[1:2, :]
    x = refx * W + sox - 0.5
    y = refy * H + soy - 0.5
    x0 = jnp.floor(x)
    y0 = jnp.floor(y)
    fx = x - x0
    fy = y - y0
    x0i = x0.astype(jnp.int32)
    y0i = y0.astype(jnp.int32)

    idx_parts = []
    w_parts = []
    for dx, dy, wc in ((0, 0, (1.0 - fx) * (1.0 - fy)),
                       (1, 0, fx * (1.0 - fy)),
                       (0, 1, (1.0 - fx) * fy),
                       (1, 1, fx * fy)):
        xi = x0i + dx
        yi = y0i + dy
        valid = (xi >= 0) & (xi <= W - 1) & (yi >= 0) & (yi <= H - 1)
        xc = jnp.clip(xi, 0, W - 1)
        yc = jnp.clip(yi, 0, H - 1)
        idx_parts.append(yc * W + xc)
        w_parts.append(wc * attn * valid.astype(jnp.float32))
    for b in range(B2H):
        sl = slice(b * POINTS, (b + 1) * POINTS)
        idx_ref[b] = jnp.concatenate([p_[sl, :] for p_ in idx_parts], axis=0)
        w_ref[b] = jnp.concatenate([p_[sl, :] for p_ in w_parts], axis=0)


def _b_body(s0_ref, s1_ref, wo_ref, bo_ref, q_ref, out_ref):
    a = jnp.concatenate([s0_ref[h] for h in range(HEADS)], axis=1)
    b = jnp.concatenate([s1_ref[h] for h in range(HEADS)], axis=1)
    m = (a + b) * 0.5
    out_ref[...] = (jnp.dot(m, wo_ref[...], preferred_element_type=jnp.float32)
                    + bo_ref[...] + q_ref[...])


def _tc_prepare(query2d, v3, refs2d, W_samp, b_samp, W_attn, b_attn,
                W_val, b_val, *, interpret=False):
    wv_t = W_val.T
    bv = b_val.reshape(1, EMBED)
    table = pl.pallas_call(
        _a1_body,
        grid=(QUEUE, NBLK),
        in_specs=[
            pl.BlockSpec((1, QB, EMBED), lambda qq, i: (qq, i, 0)),
            pl.BlockSpec((EMBED, EMBED), lambda qq, i: (0, 0)),
            pl.BlockSpec((1, EMBED), lambda qq, i: (0, 0)),
        ],
        out_specs=pl.BlockSpec((HEADS, QB, HD), lambda qq, i: (qq, i, 0)),
        out_shape=jax.ShapeDtypeStruct((B2H, NQP, HD), jnp.float32),
        interpret=interpret,
    )(v3, wv_t, bv)

    cols = jnp.arange(64, dtype=jnp.int32)
    qq_c = cols // 32
    h_c = (cols // POINTS) % HEADS
    p_c = cols % POINTS
    perm = (h_c * QUEUE + qq_c) * POINTS + p_c
    wsx = W_samp[2 * perm, :]
    wsy = W_samp[2 * perm + 1, :]
    bsx = b_samp[2 * perm].reshape(64, 1)
    bsy = b_samp[2 * perm + 1].reshape(64, 1)
    wa = W_attn[perm, :]
    ba = b_attn[perm].reshape(64, 1)
    refs_t = refs2d.T

    idx_nat, w_nat = pl.pallas_call(
        _a2_body,
        grid=(NBLK,),
        in_specs=[
            pl.BlockSpec((1, QB, EMBED), lambda i: (0, i, 0)),
            pl.BlockSpec((QB, EMBED), lambda i: (i, 0)),
            pl.BlockSpec((2, QB), lambda i: (0, i)),
            pl.BlockSpec((64, 2 * EMBED), lambda i: (0, 0)),
            pl.BlockSpec((64, 2 * EMBED), lambda i: (0, 0)),
            pl.BlockSpec((64, 1), lambda i: (0, 0)),
            pl.BlockSpec((64, 1), lambda i: (0, 0)),
            pl.BlockSpec((64, 2 * EMBED), lambda i: (0, 0)),
            pl.BlockSpec((64, 1), lambda i: (0, 0)),
        ],
        out_specs=[
            pl.BlockSpec((B2H, R, QB), lambda i: (0, 0, i)),
            pl.BlockSpec((B2H, R, QB), lambda i: (0, 0, i)),
        ],
        out_shape=[
            jax.ShapeDtypeStruct((B2H, R, NQP), jnp.int32),
            jax.ShapeDtypeStruct((B2H, R, NQP), jnp.float32),
        ],
        interpret=interpret,
    )(v3, query2d, refs_t, wsx, wsy, bsx, bsy, wa, ba)

    idx2d = idx_nat.reshape(B2H * R, NQP)
    w2d = w_nat.reshape(B2H * R, NQP)
    return table, idx2d, w2d


def _bcast_lane(vec, r):
    idx = jnp.full((LANES, 1), r, jnp.int32)
    dn = lax.GatherDimensionNumbers(offset_dims=(), collapsed_slice_dims=(0,),
                                    start_index_map=(0,))
    return lax.gather(vec, idx, dn, (1,),
                      mode=lax.GatherScatterMode.PROMISE_IN_BOUNDS)


def _sc_body(table_hbm, idx_hbm, w_hbm, out_hbm, idx_v, w_v, rows_v, out_v,
             sem0, sem1):
    cid = lax.axis_index("c")
    sid = lax.axis_index("s")
    wid = sid * NC + cid
    b2h = wid // 2
    half = wid % 2
    iota = lax.iota(jnp.int32, LANES)
    nlc = CHUNKS_PER_W
    my_table = table_hbm.at[b2h]

    def fetch_idx(c, buf):
        qpos = (half * nlc + c) * CQ
        pltpu.sync_copy(idx_hbm.at[pl.ds(b2h * R, R), pl.ds(qpos, CQ)],
                        idx_v.at[buf])
        pltpu.sync_copy(w_hbm.at[pl.ds(b2h * R, R), pl.ds(qpos, CQ)],
                        w_v.at[buf])

    def gather_copies(buf):
        sem = sem0 if buf == 0 else sem1
        return [
            pltpu.make_async_copy(my_table.at[idx_v.at[buf, r]],
                                  rows_v.at[buf, pl.ds(r * CQ, CQ)], sem)
            for r in range(R)
        ]

    fetch_idx(0, 0)
    for cp in gather_copies(0):
        cp.start()
    fetch_idx(jnp.minimum(1, nlc - 1), 1)

    def pair_body(k, carry):
        for par in (0, 1):
            c = 2 * k + par
            buf = par
            nbuf = 1 - par
            for cp in gather_copies(buf):
                cp.wait()
            for cp in gather_copies(nbuf):
                cp.start()
            rv = rows_v.at[buf]
            wvr = w_v.at[buf]

            def q_body(j, carry2):
                wv = plsc.load_gather(wvr, [iota, jnp.zeros((LANES,), jnp.int32) + j])
                acc0 = jnp.zeros((LANES,), jnp.float32)
                acc1 = jnp.zeros((LANES,), jnp.float32)
                for r in range(R):
                    wb = _bcast_lane(wv, r)
                    jr = r * CQ + j
                    acc0 = acc0 + rv[jr, pl.ds(0, LANES)] * wb
                    acc1 = acc1 + rv[jr, pl.ds(LANES, LANES)] * wb
                out_v[j, pl.ds(0, LANES)] = acc0
                out_v[j, pl.ds(LANES, LANES)] = acc1
                return carry2

            lax.fori_loop(0, CQ, q_body, 0)
            qpos = (half * nlc + c) * CQ
            pltpu.sync_copy(out_v, out_hbm.at[b2h, pl.ds(qpos, CQ)])
            fetch_idx(jnp.minimum(c + 2, nlc - 1), buf)
        return carry

    lax.fori_loop(0, nlc // 2, pair_body, 0)
    for cp in gather_copies(0):
        cp.wait()


@functools.lru_cache(maxsize=1)
def _sc_sample_fn():
    mesh = plsc.VectorSubcoreMesh(core_axis_name="c", subcore_axis_name="s")
    return pl.kernel(
        _sc_body,
        out_type=jax.ShapeDtypeStruct((B2H, NQP, HD), jnp.float32),
        mesh=mesh,
        scratch_types=[
            pltpu.VMEM((2, R, CQ), jnp.int32),
            pltpu.VMEM((2, R, CQ), jnp.float32),
            pltpu.VMEM((2, R * CQ, HD), jnp.float32),
            pltpu.VMEM((CQ, HD), jnp.float32),
            pltpu.SemaphoreType.DMA,
            pltpu.SemaphoreType.DMA,
        ],
        compiler_params=pltpu.CompilerParams(use_tc_tiling_on_sc=False,
                                             needs_layout_passes=False),
    )


def _tc_finish(out_sc, query2d, W_out, b_out, *, interpret=False):
    wo_t = W_out.T
    bo = b_out.reshape(1, EMBED)
    out = pl.pallas_call(
        _b_body,
        grid=(NBLK,),
        in_specs=[
            pl.BlockSpec((HEADS, QB, HD), lambda i: (0, i, 0)),
            pl.BlockSpec((HEADS, QB, HD), lambda i: (1, i, 0)),
            pl.BlockSpec((EMBED, EMBED), lambda i: (0, 0)),
            pl.BlockSpec((1, EMBED), lambda i: (0, 0)),
            pl.BlockSpec((QB, EMBED), lambda i: (i, 0)),
        ],
        out_specs=pl.BlockSpec((QB, EMBED), lambda i: (i, 0)),
        out_shape=jax.ShapeDtypeStruct((NQ, EMBED), jnp.float32),
        interpret=interpret,
    )(out_sc, out_sc, wo_t, bo, query2d)
    return out


def kernel(query, value, reference_points, spatial_shapes, level_start_index,
           W_samp, b_samp, W_attn, b_attn, W_val, b_val, W_out, b_out):
    query2d = query.reshape(NQ, EMBED)
    value3 = value.reshape(QUEUE, NQ, EMBED)
    refs2d = reference_points.reshape(NQ, 2)

    table, idx2d, w2d = _tc_prepare(
        query2d, value3, refs2d, W_samp, b_samp, W_attn, b_attn, W_val, b_val)

    out_sc = _sc_sample_fn()(table, idx2d, w2d)
    out = _tc_finish(out_sc, query2d, W_out, b_out)
    return out.reshape(1, NQ, EMBED)


# speedup vs baseline: 2856.0925x; 1.0122x over previous
import functools

import jax
import jax.numpy as jnp
from jax import lax
from jax.experimental import pallas as pl
from jax.experimental.pallas import tpu as pltpu
from jax.experimental.pallas import tpu_sc as plsc

H = 150
W = 150
NQ = H * W
EMBED = 256
HEADS = 8
POINTS = 4
QUEUE = 2
HD = EMBED // HEADS
B2H = QUEUE * HEADS
R = POINTS * 4

QB = 512
NQP = 44 * QB
NBLK = NQP // QB

NC = 2
NS = 16
LANES = 16
NW = NC * NS

NQH = NQP // 2
NBLK2 = NBLK // 2
CQ = 64
CHUNKS_PER_W = (NQH // CQ) // 2


def _a1_body(v_ref, wv_ref, bv_ref, tab_ref):
    y = jnp.dot(v_ref[0], wv_ref[...], preferred_element_type=jnp.float32)
    y = y + bv_ref[...]
    for h in range(HEADS):
        tab_ref[h] = y[:, h * HD:(h + 1) * HD]


def _a2_body(v0_ref, q_ref, ref_ref, wsx_ref, wsy_ref, bsx_ref, bsy_ref,
             wa_ref, ba_ref, idx_ref, w_ref):
    qe = jnp.concatenate([v0_ref[0], q_ref[...]], axis=1)
    dn = (((1,), (1,)), ((), ()))
    sox = lax.dot_general(wsx_ref[...], qe, dn,
                          preferred_element_type=jnp.float32) + bsx_ref[...]
    soy = lax.dot_general(wsy_ref[...], qe, dn,
                          preferred_element_type=jnp.float32) + bsy_ref[...]
    a = lax.dot_general(wa_ref[...], qe, dn,
                        preferred_element_type=jnp.float32) + ba_ref[...]
    s = jnp.exp(a)
    gi = lax.broadcasted_iota(jnp.int32, (64, 64), 0) // POINTS
    gj = lax.broadcasted_iota(jnp.int32, (64, 64), 1) // POINTS
    gmat = (gi == gj).astype(jnp.float32)
    denom = jnp.dot(gmat, s, preferred_element_type=jnp.float32)
    attn = s / denom

    refx = ref_ref[0:1, :]
    refy = ref_ref[1:2, :]
    x = refx * W + sox - 0.5
    y = refy * H + soy - 0.5
    x0 = jnp.floor(x)
    y0 = jnp.floor(y)
    fx = x - x0
    fy = y - y0
    x0i = x0.astype(jnp.int32)
    y0i = y0.astype(jnp.int32)

    idx_parts = []
    w_parts = []
    for dx, dy, wc in ((0, 0, (1.0 - fx) * (1.0 - fy)),
                       (1, 0, fx * (1.0 - fy)),
                       (0, 1, (1.0 - fx) * fy),
                       (1, 1, fx * fy)):
        xi = x0i + dx
        yi = y0i + dy
        valid = (xi >= 0) & (xi <= W - 1) & (yi >= 0) & (yi <= H - 1)
        xc = jnp.clip(xi, 0, W - 1)
        yc = jnp.clip(yi, 0, H - 1)
        idx_parts.append(yc * W + xc)
        w_parts.append(wc * attn * valid.astype(jnp.float32))
    for b in range(B2H):
        sl = slice(b * POINTS, (b + 1) * POINTS)
        idx_ref[b] = jnp.concatenate([p_[sl, :] for p_ in idx_parts], axis=0)
        w_ref[b] = jnp.concatenate([p_[sl, :] for p_ in w_parts], axis=0)


def _b_body(s0_ref, s1_ref, wo_ref, bo_ref, q_ref, out_ref):
    a = jnp.concatenate([s0_ref[h] for h in range(HEADS)], axis=1)
    b = jnp.concatenate([s1_ref[h] for h in range(HEADS)], axis=1)
    m = (a + b) * 0.5
    out_ref[...] = (jnp.dot(m, wo_ref[...], preferred_element_type=jnp.float32)
                    + bo_ref[...] + q_ref[...])


def _tc_prepare(query2d, v3, refs2d, W_samp, b_samp, W_attn, b_attn,
                W_val, b_val, *, interpret=False):
    wv_t = W_val.T
    bv = b_val.reshape(1, EMBED)
    table = pl.pallas_call(
        _a1_body,
        grid=(QUEUE, NBLK),
        in_specs=[
            pl.BlockSpec((1, QB, EMBED), lambda qq, i: (qq, i, 0)),
            pl.BlockSpec((EMBED, EMBED), lambda qq, i: (0, 0)),
            pl.BlockSpec((1, EMBED), lambda qq, i: (0, 0)),
        ],
        out_specs=pl.BlockSpec((HEADS, QB, HD), lambda qq, i: (qq, i, 0)),
        out_shape=jax.ShapeDtypeStruct((B2H, NQP, HD), jnp.float32),
        interpret=interpret,
    )(v3, wv_t, bv)

    cols = jnp.arange(64, dtype=jnp.int32)
    qq_c = cols // 32
    h_c = (cols // POINTS) % HEADS
    p_c = cols % POINTS
    perm = (h_c * QUEUE + qq_c) * POINTS + p_c
    wsx = W_samp[2 * perm, :]
    wsy = W_samp[2 * perm + 1, :]
    bsx = b_samp[2 * perm].reshape(64, 1)
    bsy = b_samp[2 * perm + 1].reshape(64, 1)
    wa = W_attn[perm, :]
    ba = b_attn[perm].reshape(64, 1)
    refs_t = refs2d.T

    halves = []
    for h in (0, 1):
        off = h * NBLK2
        idx_nat, w_nat = pl.pallas_call(
            _a2_body,
            grid=(NBLK2,),
            in_specs=[
                pl.BlockSpec((1, QB, EMBED), lambda i, o=off: (0, i + o, 0)),
                pl.BlockSpec((QB, EMBED), lambda i, o=off: (i + o, 0)),
                pl.BlockSpec((2, QB), lambda i, o=off: (0, i + o)),
                pl.BlockSpec((64, 2 * EMBED), lambda i: (0, 0)),
                pl.BlockSpec((64, 2 * EMBED), lambda i: (0, 0)),
                pl.BlockSpec((64, 1), lambda i: (0, 0)),
                pl.BlockSpec((64, 1), lambda i: (0, 0)),
                pl.BlockSpec((64, 2 * EMBED), lambda i: (0, 0)),
                pl.BlockSpec((64, 1), lambda i: (0, 0)),
            ],
            out_specs=[
                pl.BlockSpec((B2H, R, QB), lambda i: (0, 0, i)),
                pl.BlockSpec((B2H, R, QB), lambda i: (0, 0, i)),
            ],
            out_shape=[
                jax.ShapeDtypeStruct((B2H, R, NQH), jnp.int32),
                jax.ShapeDtypeStruct((B2H, R, NQH), jnp.float32),
            ],
            interpret=interpret,
        )(v3, query2d, refs_t, wsx, wsy, bsx, bsy, wa, ba)
        halves.append((idx_nat.reshape(B2H * R, NQH),
                       w_nat.reshape(B2H * R, NQH)))
    return table, halves


def _bcast_lane(vec, r):
    idx = jnp.full((LANES, 1), r, jnp.int32)
    dn = lax.GatherDimensionNumbers(offset_dims=(), collapsed_slice_dims=(0,),
                                    start_index_map=(0,))
    return lax.gather(vec, idx, dn, (1,),
                      mode=lax.GatherScatterMode.PROMISE_IN_BOUNDS)


def _sc_body(table_hbm, idx_hbm, w_hbm, out_hbm, idx_v, w_v, rows_v, out_v,
             sem0, sem1):
    cid = lax.axis_index("c")
    sid = lax.axis_index("s")
    wid = sid * NC + cid
    b2h = wid // 2
    half = wid % 2
    iota = lax.iota(jnp.int32, LANES)
    nlc = CHUNKS_PER_W
    my_table = table_hbm.at[b2h]

    def fetch_idx(c, buf):
        qpos = (half * nlc + c) * CQ
        pltpu.sync_copy(idx_hbm.at[pl.ds(b2h * R, R), pl.ds(qpos, CQ)],
                        idx_v.at[buf])
        pltpu.sync_copy(w_hbm.at[pl.ds(b2h * R, R), pl.ds(qpos, CQ)],
                        w_v.at[buf])

    def gather_copies(buf):
        sem = sem0 if buf == 0 else sem1
        return [
            pltpu.make_async_copy(my_table.at[idx_v.at[buf, r]],
                                  rows_v.at[buf, pl.ds(r * CQ, CQ)], sem)
            for r in range(R)
        ]

    fetch_idx(0, 0)
    for cp in gather_copies(0):
        cp.start()
    fetch_idx(jnp.minimum(1, nlc - 1), 1)

    def pair_body(k, carry):
        for par in (0, 1):
            c = 2 * k + par
            buf = par
            nbuf = 1 - par
            for cp in gather_copies(buf):
                cp.wait()
            for cp in gather_copies(nbuf):
                cp.start()
            rv = rows_v.at[buf]
            wvr = w_v.at[buf]

            def q_body(j, carry2):
                wv = plsc.load_gather(wvr, [iota, jnp.zeros((LANES,), jnp.int32) + j])
                acc0 = jnp.zeros((LANES,), jnp.float32)
                acc1 = jnp.zeros((LANES,), jnp.float32)
                for r in range(R):
                    wb = _bcast_lane(wv, r)
                    jr = r * CQ + j
                    acc0 = acc0 + rv[jr, pl.ds(0, LANES)] * wb
                    acc1 = acc1 + rv[jr, pl.ds(LANES, LANES)] * wb
                out_v[j, pl.ds(0, LANES)] = acc0
                out_v[j, pl.ds(LANES, LANES)] = acc1
                return carry2

            lax.fori_loop(0, CQ, q_body, 0)
            qpos = (half * nlc + c) * CQ
            pltpu.sync_copy(out_v, out_hbm.at[b2h, pl.ds(qpos, CQ)])
            fetch_idx(jnp.minimum(c + 2, nlc - 1), buf)
        return carry

    lax.fori_loop(0, nlc // 2, pair_body, 0)
    for cp in gather_copies(0):
        cp.wait()


@functools.lru_cache(maxsize=1)
def _sc_sample_fn():
    mesh = plsc.VectorSubcoreMesh(core_axis_name="c", subcore_axis_name="s")
    return pl.kernel(
        _sc_body,
        out_type=jax.ShapeDtypeStruct((B2H, NQH, HD), jnp.float32),
        mesh=mesh,
        scratch_types=[
            pltpu.VMEM((2, R, CQ), jnp.int32),
            pltpu.VMEM((2, R, CQ), jnp.float32),
            pltpu.VMEM((2, R * CQ, HD), jnp.float32),
            pltpu.VMEM((CQ, HD), jnp.float32),
            pltpu.SemaphoreType.DMA,
            pltpu.SemaphoreType.DMA,
        ],
        compiler_params=pltpu.CompilerParams(use_tc_tiling_on_sc=False,
                                             needs_layout_passes=False),
    )


def _tc_finish(out_sc_h, query2d, W_out, b_out, h, *, interpret=False):
    wo_t = W_out.T
    bo = b_out.reshape(1, EMBED)
    off = h * NBLK2
    nrows = NQH if h == 0 else NQ - NQH
    out = pl.pallas_call(
        _b_body,
        grid=(NBLK2,),
        in_specs=[
            pl.BlockSpec((HEADS, QB, HD), lambda i: (0, i, 0)),
            pl.BlockSpec((HEADS, QB, HD), lambda i: (1, i, 0)),
            pl.BlockSpec((EMBED, EMBED), lambda i: (0, 0)),
            pl.BlockSpec((1, EMBED), lambda i: (0, 0)),
            pl.BlockSpec((QB, EMBED), lambda i, o=off: (i + o, 0)),
        ],
        out_specs=pl.BlockSpec((QB, EMBED), lambda i: (i, 0)),
        out_shape=jax.ShapeDtypeStruct((nrows, EMBED), jnp.float32),
        interpret=interpret,
    )(out_sc_h, out_sc_h, wo_t, bo, query2d)
    return out


def kernel(query, value, reference_points, spatial_shapes, level_start_index,
           W_samp, b_samp, W_attn, b_attn, W_val, b_val, W_out, b_out):
    query2d = query.reshape(NQ, EMBED)
    value3 = value.reshape(QUEUE, NQ, EMBED)
    refs2d = reference_points.reshape(NQ, 2)

    table, halves = _tc_prepare(
        query2d, value3, refs2d, W_samp, b_samp, W_attn, b_attn, W_val, b_val)

    sc = _sc_sample_fn()
    outs = []
    for h, (idx2d, w2d) in enumerate(halves):
        out_sc_h = sc(table, idx2d, w2d)
        outs.append(_tc_finish(out_sc_h, query2d, W_out, b_out, h))
    return jnp.concatenate(outs, axis=0).reshape(1, NQ, EMBED)
